# speedup vs baseline: 2.9722x; 2.9722x over previous
import functools

import jax
import jax.numpy as jnp
from jax import lax
from jax.experimental import pallas as pl
from jax.experimental.pallas import tpu as pltpu
from jax.experimental.pallas import tpu_sc as plsc

_NUM_CORES = 2
_NUM_SUBCORES = 16
_NW = _NUM_CORES * _NUM_SUBCORES
_CHUNK = 128


def _gather_body(n_chunks, idx_hbm, w_hbm, out_hbm, idx_v, rows_v, sem):
    wid = lax.axis_index("s") * _NUM_CORES + lax.axis_index("c")
    pltpu.sync_copy(idx_hbm.at[wid], idx_v)

    def chunk(c, carry):
        cp = pltpu.async_copy(w_hbm.at[idx_v.at[c]], rows_v, sem)
        cp.wait()
        base = (wid * n_chunks + c) * _CHUNK
        pltpu.sync_copy(rows_v, out_hbm.at[pl.ds(base, _CHUNK)])
        return carry

    lax.fori_loop(0, n_chunks, chunk, 0)


@functools.partial(jax.jit, static_argnums=(2, 3))
def _embedding_lookup(idx, w, n_chunks, d):
    mesh = plsc.VectorSubcoreMesh(core_axis_name="c", subcore_axis_name="s")
    out = pl.kernel(
        functools.partial(_gather_body, n_chunks),
        out_type=jax.ShapeDtypeStruct((_NW * n_chunks * _CHUNK, d), w.dtype),
        mesh=mesh,
        scratch_types=[
            pltpu.VMEM((n_chunks, _CHUNK), jnp.int32),
            pltpu.VMEM((_CHUNK, d), w.dtype),
            pltpu.SemaphoreType.DMA,
        ],
    )(idx, w)
    return out


def kernel(token_ids, W):
    b, h = token_ids.shape
    n, d = W.shape
    total = b * h
    assert total % (_NW * _CHUNK) == 0
    n_chunks = total // (_NW * _CHUNK)
    idx = token_ids.reshape(_NW, n_chunks, _CHUNK).astype(jnp.int32)
    out = _embedding_lookup(idx, W, n_chunks, d)
    return out.reshape(b, h, d)


# speedup vs baseline: 3.3501x; 1.1272x over previous
import functools

import jax
import jax.numpy as jnp
from jax import lax
from jax.experimental import pallas as pl
from jax.experimental.pallas import tpu as pltpu
from jax.experimental.pallas import tpu_sc as plsc

_NUM_CORES = 2
_NUM_SUBCORES = 16
_NW = _NUM_CORES * _NUM_SUBCORES
_CHUNK = 128


_NBUF = 5


def _gather_body(n_chunks, idx_hbm, w_hbm, out_hbm, idx_v, rows_v, sems):
    wid = lax.axis_index("s") * _NUM_CORES + lax.axis_index("c")
    pltpu.sync_copy(idx_hbm.at[wid], idx_v)

    for b in range(_NBUF):
        pltpu.async_copy(w_hbm.at[idx_v.at[b]], rows_v.at[b], sems.at[b])

    def group(g, carry):
        for b in range(_NBUF):
            c = g * _NBUF + b
            pltpu.make_async_copy(
                w_hbm.at[idx_v.at[c]], rows_v.at[b], sems.at[b]
            ).wait()
            base = (wid * n_chunks + c) * _CHUNK
            pltpu.sync_copy(rows_v.at[b], out_hbm.at[pl.ds(base, _CHUNK)])

            @pl.when(c + _NBUF < n_chunks)
            def _():
                pltpu.async_copy(
                    w_hbm.at[idx_v.at[c + _NBUF]], rows_v.at[b], sems.at[b]
                )

        return carry

    lax.fori_loop(0, n_chunks // _NBUF, group, 0)


@functools.partial(jax.jit, static_argnums=(2, 3))
def _embedding_lookup(idx, w, n_chunks, d):
    mesh = plsc.VectorSubcoreMesh(core_axis_name="c", subcore_axis_name="s")
    out = pl.kernel(
        functools.partial(_gather_body, n_chunks),
        out_type=jax.ShapeDtypeStruct((_NW * n_chunks * _CHUNK, d), w.dtype),
        mesh=mesh,
        scratch_types=[
            pltpu.VMEM((n_chunks, _CHUNK), jnp.int32),
            pltpu.VMEM((_NBUF, _CHUNK, d), w.dtype),
            pltpu.SemaphoreType.DMA((_NBUF,)),
        ],
    )(idx, w)
    return out


def kernel(token_ids, W):
    b, h = token_ids.shape
    n, d = W.shape
    total = b * h
    assert total % (_NW * _CHUNK) == 0
    n_chunks = total // (_NW * _CHUNK)
    assert n_chunks % _NBUF == 0
    idx = token_ids.reshape(_NW, n_chunks, _CHUNK).astype(jnp.int32)
    out = _embedding_lookup(idx, W, n_chunks, d)
    return out.reshape(b, h, d)


# speedup vs baseline: 5.9783x; 1.7845x over previous
import functools

import jax
import jax.numpy as jnp
from jax import lax
from jax.experimental import pallas as pl
from jax.experimental.pallas import tpu as pltpu
from jax.experimental.pallas import tpu_sc as plsc

_NUM_CORES = 2
_NUM_SUBCORES = 16
_NW = _NUM_CORES * _NUM_SUBCORES
_NB = 2
_NBUF = 4


def _gather_body(n_chunks, h, idx_hbm, w_hbm, out_hbm, idx_v, rows_v, sems):
    wid = lax.axis_index("s") * _NUM_CORES + lax.axis_index("c")
    pltpu.sync_copy(idx_hbm.at[wid], idx_v)
    batch0 = wid * (n_chunks * _NB)

    for b in range(_NBUF):
        pltpu.async_copy(w_hbm.at[idx_v.at[b]], rows_v.at[b], sems.at[b])

    def group(g, carry):
        for b in range(_NBUF):
            c = g * _NBUF + b
            pltpu.make_async_copy(
                w_hbm.at[idx_v.at[c]], rows_v.at[b], sems.at[b]
            ).wait()
            for s in range(_NB):
                pltpu.sync_copy(
                    rows_v.at[b].at[pl.ds(s * h, h)],
                    out_hbm.at[batch0 + c * _NB + s],
                )

            @pl.when(c + _NBUF < n_chunks)
            def _():
                pltpu.async_copy(
                    w_hbm.at[idx_v.at[c + _NBUF]], rows_v.at[b], sems.at[b]
                )

        return carry

    lax.fori_loop(0, n_chunks // _NBUF, group, 0)


@functools.partial(jax.jit, static_argnums=(2, 3, 4))
def _embedding_lookup(idx, w, n_chunks, h, d):
    mesh = plsc.VectorSubcoreMesh(core_axis_name="c", subcore_axis_name="s")
    out = pl.kernel(
        functools.partial(_gather_body, n_chunks, h),
        out_type=jax.ShapeDtypeStruct((_NW * n_chunks * _NB, h, d), w.dtype),
        mesh=mesh,
        scratch_types=[
            pltpu.VMEM((n_chunks, _NB * h), jnp.int32),
            pltpu.VMEM((_NBUF, _NB * h, d), w.dtype),
            pltpu.SemaphoreType.DMA((_NBUF,)),
        ],
    )(idx, w)
    return out


def kernel(token_ids, W):
    b, h = token_ids.shape
    n, d = W.shape
    assert b % (_NW * _NB) == 0
    n_chunks = b // (_NW * _NB)
    assert n_chunks % _NBUF == 0
    idx = token_ids.reshape(_NW, n_chunks, _NB * h).astype(jnp.int32)
    return _embedding_lookup(idx, W, n_chunks, h, d)
